# Initial kernel scaffold; baseline (speedup 1.0000x reference)
import jax
import jax.numpy as jnp
from jax.experimental import pallas as pl


def kernel(h, edge_index, e):
    raise NotImplementedError("write your pallas kernel here")


# speedup vs baseline: 5.4363x; 5.4363x over previous
import functools

import jax
import jax.numpy as jnp
from jax import lax
from jax.experimental import pallas as pl
from jax.experimental.pallas import tpu as pltpu
from jax.experimental.pallas import tpu_sc as plsc

N_NODES = 10000
N_EDGES = 320000
D_EDGE = 16
CHUNK = 128
N_CHUNKS = N_EDGES // CHUNK
NW = 32
PER_TILE = N_CHUNKS // NW
LEFTOVER = N_CHUNKS - PER_TILE * NW
CB = 13
NB = PER_TILE // CB
NSUB = 16


def _sc_segment_sum(dst3d, e, zeros_hbm):
    mesh = plsc.VectorSubcoreMesh(core_axis_name="c", subcore_axis_name="s")

    @functools.partial(
        pl.kernel,
        mesh=mesh,
        out_type=jax.ShapeDtypeStruct((2, N_NODES, D_EDGE), jnp.float32),
        compiler_params=pltpu.CompilerParams(use_tc_tiling_on_sc=False),
        scratch_types=[
            pltpu.VMEM((CB, 1, CHUNK), jnp.int32),
            pltpu.VMEM((CB * CHUNK, D_EDGE), jnp.float32),
            pltpu.VMEM_SHARED((N_NODES, D_EDGE), jnp.float32),
        ],
    )
    def k(dst_hbm, e_hbm, z_hbm, out_hbm, idx_v, e_v, acc_sh):
        cid = lax.axis_index("c")
        sid = lax.axis_index("s")
        wid = sid * 2 + cid

        @pl.when(sid == 0)
        def _():
            pltpu.sync_copy(z_hbm, acc_sh)

        plsc.subcore_barrier()

        c0 = wid * PER_TILE

        def block(b, carry):
            cb = c0 + b * CB
            pltpu.sync_copy(dst_hbm.at[pl.ds(cb, CB)], idx_v)
            pltpu.sync_copy(e_hbm.at[pl.ds(cb * CHUNK, CB * CHUNK)], e_v)
            for j in range(CB):
                pltpu.sync_copy(
                    e_v.at[pl.ds(j * CHUNK, CHUNK)],
                    acc_sh.at[idx_v.at[j, 0]],
                    add=True,
                )
            return carry

        lax.fori_loop(0, NB, block, 0)

        @pl.when(wid < LEFTOVER)
        def _():
            c = NW * PER_TILE + wid
            pltpu.sync_copy(dst_hbm.at[pl.ds(c, 1)], idx_v.at[pl.ds(0, 1)])
            pltpu.sync_copy(e_hbm.at[pl.ds(c * CHUNK, CHUNK)],
                            e_v.at[pl.ds(0, CHUNK)])
            pltpu.sync_copy(e_v.at[pl.ds(0, CHUNK)],
                            acc_sh.at[idx_v.at[0, 0]], add=True)

        plsc.subcore_barrier()

        @pl.when(sid == 0)
        def _():
            pltpu.sync_copy(acc_sh, out_hbm.at[cid])

    return k(dst3d, e, zeros_hbm)


def _combine_body(p_ref, o_ref):
    o_ref[...] = p_ref[0] + p_ref[1]


def kernel(h, edge_index, e):
    del h
    dst3d = edge_index[1].astype(jnp.int32).reshape(N_CHUNKS, 1, CHUNK)
    zeros_hbm = jnp.zeros((N_NODES, D_EDGE), jnp.float32)
    parts = _sc_segment_sum(dst3d, e, zeros_hbm)
    p = parts.reshape(2, (N_NODES * D_EDGE) // 128, 128)
    out = pl.pallas_call(
        _combine_body,
        out_shape=jax.ShapeDtypeStruct(((N_NODES * D_EDGE) // 128, 128),
                                       jnp.float32),
    )(p)
    return out.reshape(N_NODES, D_EDGE)


# speedup vs baseline: 5.6374x; 1.0370x over previous
import functools

import jax
import jax.numpy as jnp
from jax import lax
from jax.experimental import pallas as pl
from jax.experimental.pallas import tpu as pltpu
from jax.experimental.pallas import tpu_sc as plsc

N_NODES = 10000
N_EDGES = 320000
D_EDGE = 16
CHUNK = 128
N_CHUNKS = N_EDGES // CHUNK
HALVES = 2
CPH = N_CHUNKS // HALVES
CBLK = 25
NBLK = CPH // CBLK
LANES = 16


def _sc_segment_sum(e4, dst, zeros_hbm):
    mesh = plsc.VectorSubcoreMesh(core_axis_name="c", subcore_axis_name="s")

    @functools.partial(
        pl.kernel,
        mesh=mesh,
        out_type=jax.ShapeDtypeStruct((HALVES, D_EDGE, N_NODES), jnp.float32),
        compiler_params=pltpu.CompilerParams(use_tc_tiling_on_sc=False,
                                             needs_layout_passes=False),
        scratch_types=[
            pltpu.VMEM((CBLK * CHUNK,), jnp.int32),
            pltpu.VMEM((CBLK, 1, CHUNK), jnp.float32),
            pltpu.VMEM((N_NODES,), jnp.float32),
        ],
    )
    def k(e_hbm, dst_hbm, z_hbm, out_hbm, idx_v, val_v, acc_v):
        cid = lax.axis_index("c")
        sid = lax.axis_index("s")
        wid = sid * 2 + cid
        f = wid // 2
        h = wid % 2
        g = f // 8
        r = f % 8

        pltpu.sync_copy(z_hbm, acc_v)

        def block(b, carry):
            c0 = h * CPH + b * CBLK
            pltpu.sync_copy(dst_hbm.at[pl.ds(c0 * CHUNK, CBLK * CHUNK)],
                            idx_v)
            pltpu.sync_copy(e_hbm.at[g, pl.ds(c0, CBLK), pl.ds(r, 1)],
                            val_v)

            def chunk_step(c, carry2):
                base = c * CHUNK
                for t in range(CHUNK // LANES):
                    i16 = idx_v[pl.ds(base + t * LANES, LANES)]
                    v16 = val_v[c, 0, pl.ds(t * LANES, LANES)]
                    plsc.addupdate_scatter(acc_v, [i16], v16)
                return carry2

            lax.fori_loop(0, CBLK, chunk_step, 0)
            return carry

        lax.fori_loop(0, NBLK, block, 0)

        pltpu.sync_copy(acc_v, out_hbm.at[h, f])

    return k(e4, dst, zeros_hbm)


def _combine_body(p_ref, o_ref):
    o_ref[...] = p_ref[0] + p_ref[1]


def kernel(h, edge_index, e):
    del h
    dst = edge_index[1].astype(jnp.int32)
    e4 = e.T.reshape(2, 8, N_CHUNKS, CHUNK).transpose(0, 2, 1, 3)
    zeros_hbm = jnp.zeros((N_NODES,), jnp.float32)
    parts = _sc_segment_sum(e4, dst, zeros_hbm)
    out_t = pl.pallas_call(
        _combine_body,
        out_shape=jax.ShapeDtypeStruct((D_EDGE, N_NODES), jnp.float32),
    )(parts)
    return out_t.T


# speedup vs baseline: 15.0991x; 2.6784x over previous
import functools

import jax
import jax.numpy as jnp
from jax import lax
from jax.experimental import pallas as pl
from jax.experimental.pallas import tpu as pltpu
from jax.experimental.pallas import tpu_sc as plsc

N_NODES = 10000
N_EDGES = 320000
D_EDGE = 16
CHUNK = 128
N_CHUNKS = N_EDGES // CHUNK
HALVES = 2
CPH = N_CHUNKS // HALVES
CBLK = 125
NBLK = CPH // CBLK
LANES = 16


def _sc_segment_sum(e4, dst, zeros_hbm):
    mesh = plsc.VectorSubcoreMesh(core_axis_name="c", subcore_axis_name="s")

    @functools.partial(
        pl.kernel,
        mesh=mesh,
        out_type=jax.ShapeDtypeStruct((HALVES, D_EDGE, N_NODES), jnp.float32),
        compiler_params=pltpu.CompilerParams(use_tc_tiling_on_sc=False,
                                             needs_layout_passes=False),
        scratch_types=[
            pltpu.VMEM((CBLK * CHUNK,), jnp.int32),
            pltpu.VMEM((CBLK * CHUNK,), jnp.int32),
            pltpu.VMEM((CBLK, 1, CHUNK), jnp.float32),
            pltpu.VMEM((CBLK, 1, CHUNK), jnp.float32),
            pltpu.VMEM((N_NODES,), jnp.float32),
            pltpu.SemaphoreType.DMA,
            pltpu.SemaphoreType.DMA,
        ],
    )
    def k(e_hbm, dst_hbm, z_hbm, out_hbm,
          idx_v0, idx_v1, val_v0, val_v1, acc_v, sem0, sem1):
        cid = lax.axis_index("c")
        sid = lax.axis_index("s")
        wid = sid * 2 + cid
        f = wid // 2
        h = wid % 2
        g = f // 8
        r = f % 8

        idx_bufs = (idx_v0, idx_v1)
        val_bufs = (val_v0, val_v1)
        sems = (sem0, sem1)

        def start(b):
            c0 = h * CPH + b * CBLK
            i = b % 2
            cp1 = pltpu.async_copy(
                dst_hbm.at[pl.ds(c0 * CHUNK, CBLK * CHUNK)],
                idx_bufs[i], sems[i])
            cp2 = pltpu.async_copy(
                e_hbm.at[g, pl.ds(c0, CBLK), pl.ds(r, 1)],
                val_bufs[i], sems[i])
            return cp1, cp2

        pltpu.sync_copy(z_hbm, acc_v)
        cps = start(0)
        for b in range(NBLK):
            cp1, cp2 = cps
            cp1.wait()
            cp2.wait()
            if b + 1 < NBLK:
                cps = start(b + 1)
            iv = idx_bufs[b % 2]
            vv = val_bufs[b % 2]

            @functools.partial(plsc.parallel_loop, 0, CBLK, unroll=4)
            def _(c):
                base = c * CHUNK
                for t in range(CHUNK // LANES):
                    i16 = iv[pl.ds(base + t * LANES, LANES)]
                    v16 = vv[c, 0, pl.ds(t * LANES, LANES)]
                    plsc.addupdate_scatter(acc_v, [i16], v16)

        pltpu.sync_copy(acc_v, out_hbm.at[h, f])

    return k(e4, dst, zeros_hbm)


def _combine_body(p_ref, o_ref):
    o_ref[...] = p_ref[0] + p_ref[1]


def kernel(h, edge_index, e):
    del h
    dst = edge_index[1].astype(jnp.int32)
    e4 = e.T.reshape(2, 8, N_CHUNKS, CHUNK).transpose(0, 2, 1, 3)
    zeros_hbm = jnp.zeros((N_NODES,), jnp.float32)
    parts = _sc_segment_sum(e4, dst, zeros_hbm)
    out_t = pl.pallas_call(
        _combine_body,
        out_shape=jax.ShapeDtypeStruct((D_EDGE, N_NODES), jnp.float32),
    )(parts)
    return out_t.T
